# initial kernel scaffold (unmeasured)
import jax
import jax.numpy as jnp
from jax import lax
from jax.experimental import pallas as pl
from jax.experimental.pallas import tpu as pltpu


def kernel(
    u,
):
    def body(*refs):
        pass

    out_shape = jax.ShapeDtypeStruct(..., jnp.float32)
    return pl.pallas_call(body, out_shape=out_shape)(...)


# baseline (device time: 7540 ns/iter reference)
import jax
import jax.numpy as jnp
from jax import lax
from jax.experimental import pallas as pl
from jax.experimental.pallas import tpu as pltpu

S = 16


def kernel(u):
    assert u.shape == (S, S, S), u.shape

    def body(u_ref, out_ref, pad_ref, fsend, frecv, send_sems, recv_sems):
        my = [lax.axis_index("x"), lax.axis_index("y"), lax.axis_index("z")]

        def neighbor(a):
            nbr = [my[0], my[1], my[2]]
            nbr[a] = 1 - nbr[a]
            return (nbr[0], nbr[1], nbr[2])

        barrier = pltpu.get_barrier_semaphore()
        for a in range(3):
            pl.semaphore_signal(
                barrier,
                inc=1,
                device_id=neighbor(a),
                device_id_type=pl.DeviceIdType.MESH,
            )
        pl.semaphore_wait(barrier, 3)

        @pl.when(my[0] == 0)
        def _():
            fsend[0, :, :] = u_ref[S - 1, :, :]

        @pl.when(my[0] == 1)
        def _():
            fsend[0, :, :] = u_ref[0, :, :]

        @pl.when(my[1] == 0)
        def _():
            fsend[1, :, :] = u_ref[:, S - 1, :]

        @pl.when(my[1] == 1)
        def _():
            fsend[1, :, :] = u_ref[:, 0, :]

        @pl.when(my[2] == 0)
        def _():
            fsend[2, :, :] = u_ref[:, :, S - 1]

        @pl.when(my[2] == 1)
        def _():
            fsend[2, :, :] = u_ref[:, :, 0]

        rdmas = []
        for a in range(3):
            rdma = pltpu.make_async_remote_copy(
                src_ref=fsend.at[a],
                dst_ref=frecv.at[a],
                send_sem=send_sems.at[a],
                recv_sem=recv_sems.at[a],
                device_id=neighbor(a),
                device_id_type=pl.DeviceIdType.MESH,
            )
            rdma.start()
            rdmas.append(rdma)

        pad_ref[:, :, :] = jnp.zeros((S + 2, S + 2, S + 2), jnp.float32)
        pad_ref[1 : S + 1, 1 : S + 1, 1 : S + 1] = u_ref[:, :, :]

        for r in rdmas:
            r.wait()

        @pl.when(my[0] == 0)
        def _():
            pad_ref[S + 1, 1 : S + 1, 1 : S + 1] = frecv[0, :, :]

        @pl.when(my[0] == 1)
        def _():
            pad_ref[0, 1 : S + 1, 1 : S + 1] = frecv[0, :, :]

        @pl.when(my[1] == 0)
        def _():
            pad_ref[1 : S + 1, S + 1, 1 : S + 1] = frecv[1, :, :]

        @pl.when(my[1] == 1)
        def _():
            pad_ref[1 : S + 1, 0, 1 : S + 1] = frecv[1, :, :]

        @pl.when(my[2] == 0)
        def _():
            pad_ref[1 : S + 1, 1 : S + 1, S + 1] = frecv[2, :, :]

        @pl.when(my[2] == 1)
        def _():
            pad_ref[1 : S + 1, 1 : S + 1, 0] = frecv[2, :, :]

        p = pad_ref[:, :, :]
        c = p[1 : S + 1, 1 : S + 1, 1 : S + 1]
        v = (
            p[0:S, 1 : S + 1, 1 : S + 1]
            + p[2 : S + 2, 1 : S + 1, 1 : S + 1]
            + p[1 : S + 1, 0:S, 1 : S + 1]
            + p[1 : S + 1, 2 : S + 2, 1 : S + 1]
            + p[1 : S + 1, 1 : S + 1, 0:S]
            + p[1 : S + 1, 1 : S + 1, 2 : S + 2]
            - 6.0 * c
        )

        bnd = jnp.zeros((S, S, S), jnp.bool_)
        for a in range(3):
            idx = lax.broadcasted_iota(jnp.int32, (S, S, S), a)
            bnd = bnd | (idx == my[a] * (S - 1))
        out_ref[:, :, :] = jnp.where(bnd, 0.0, v)

    return pl.pallas_call(
        body,
        out_shape=jax.ShapeDtypeStruct((S, S, S), jnp.float32),
        in_specs=[pl.BlockSpec(memory_space=pltpu.VMEM)],
        out_specs=pl.BlockSpec(memory_space=pltpu.VMEM),
        scratch_shapes=[
            pltpu.VMEM((S + 2, S + 2, S + 2), jnp.float32),
            pltpu.VMEM((3, S, S), jnp.float32),
            pltpu.VMEM((3, S, S), jnp.float32),
            pltpu.SemaphoreType.DMA((3,)),
            pltpu.SemaphoreType.DMA((3,)),
        ],
        compiler_params=pltpu.CompilerParams(collective_id=0),
    )(u)


# device time: 7011 ns/iter; 1.0755x vs baseline; 1.0755x over previous
import jax
import jax.numpy as jnp
from jax import lax
from jax.experimental import pallas as pl
from jax.experimental.pallas import tpu as pltpu

S = 16


def kernel(u):
    assert u.shape == (S, S, S), u.shape

    def body(u_ref, out_ref, fsend, frecv, send_sems, recv_sems):
        my = [lax.axis_index("x"), lax.axis_index("y"), lax.axis_index("z")]

        def neighbor(a):
            nbr = [my[0], my[1], my[2]]
            nbr[a] = 1 - nbr[a]
            return (nbr[0], nbr[1], nbr[2])

        barrier = pltpu.get_barrier_semaphore()
        for a in range(3):
            pl.semaphore_signal(
                barrier,
                inc=1,
                device_id=neighbor(a),
                device_id_type=pl.DeviceIdType.MESH,
            )

        @pl.when(my[0] == 0)
        def _():
            fsend[0, :, :] = u_ref[S - 1, :, :]

        @pl.when(my[0] == 1)
        def _():
            fsend[0, :, :] = u_ref[0, :, :]

        @pl.when(my[1] == 0)
        def _():
            fsend[1, :, :] = u_ref[:, S - 1, :]

        @pl.when(my[1] == 1)
        def _():
            fsend[1, :, :] = u_ref[:, 0, :]

        @pl.when(my[2] == 0)
        def _():
            fsend[2, :, :] = u_ref[:, :, S - 1]

        @pl.when(my[2] == 1)
        def _():
            fsend[2, :, :] = u_ref[:, :, 0]

        pl.semaphore_wait(barrier, 3)

        rdmas = []
        for a in range(3):
            rdma = pltpu.make_async_remote_copy(
                src_ref=fsend.at[a],
                dst_ref=frecv.at[a],
                send_sem=send_sems.at[a],
                recv_sem=recv_sems.at[a],
                device_id=neighbor(a),
                device_id_type=pl.DeviceIdType.MESH,
            )
            rdma.start()
            rdmas.append(rdma)

        u_val = u_ref[:, :, :]
        z0 = jnp.zeros((1, S, S), jnp.float32)
        z1 = jnp.zeros((S, 1, S), jnp.float32)
        z2 = jnp.zeros((S, S, 1), jnp.float32)
        v = (
            jnp.concatenate([u_val[1:], z0], 0)
            + jnp.concatenate([z0, u_val[:-1]], 0)
            + jnp.concatenate([u_val[:, 1:], z1], 1)
            + jnp.concatenate([z1, u_val[:, :-1]], 1)
            + jnp.concatenate([u_val[:, :, 1:], z2], 2)
            + jnp.concatenate([z2, u_val[:, :, :-1]], 2)
            - 6.0 * u_val
        )
        bnd = jnp.zeros((S, S, S), jnp.bool_)
        for a in range(3):
            idx = lax.broadcasted_iota(jnp.int32, (S, S, S), a)
            bnd = bnd | (idx == my[a] * (S - 1))
        out_ref[:, :, :] = jnp.where(bnd, 0.0, v)

        def plane_mask(b, c):
            ib = lax.broadcasted_iota(jnp.int32, (S, S), 0)
            ic = lax.broadcasted_iota(jnp.int32, (S, S), 1)
            return (ib == my[b] * (S - 1)) | (ic == my[c] * (S - 1))

        rdmas[0].wait_recv()
        h0 = jnp.where(plane_mask(1, 2), 0.0, frecv[0, :, :])

        @pl.when(my[0] == 0)
        def _():
            out_ref[S - 1, :, :] = out_ref[S - 1, :, :] + h0

        @pl.when(my[0] == 1)
        def _():
            out_ref[0, :, :] = out_ref[0, :, :] + h0

        rdmas[1].wait_recv()
        h1 = jnp.where(plane_mask(0, 2), 0.0, frecv[1, :, :])

        @pl.when(my[1] == 0)
        def _():
            out_ref[:, S - 1, :] = out_ref[:, S - 1, :] + h1

        @pl.when(my[1] == 1)
        def _():
            out_ref[:, 0, :] = out_ref[:, 0, :] + h1

        rdmas[2].wait_recv()
        h2 = jnp.where(plane_mask(0, 1), 0.0, frecv[2, :, :])

        @pl.when(my[2] == 0)
        def _():
            out_ref[:, :, S - 1] = out_ref[:, :, S - 1] + h2

        @pl.when(my[2] == 1)
        def _():
            out_ref[:, :, 0] = out_ref[:, :, 0] + h2

        for r in rdmas:
            r.wait_send()

    return pl.pallas_call(
        body,
        out_shape=jax.ShapeDtypeStruct((S, S, S), jnp.float32),
        in_specs=[pl.BlockSpec(memory_space=pltpu.VMEM)],
        out_specs=pl.BlockSpec(memory_space=pltpu.VMEM),
        scratch_shapes=[
            pltpu.VMEM((3, S, S), jnp.float32),
            pltpu.VMEM((3, S, S), jnp.float32),
            pltpu.SemaphoreType.DMA((3,)),
            pltpu.SemaphoreType.DMA((3,)),
        ],
        compiler_params=pltpu.CompilerParams(collective_id=0),
    )(u)
